# baseline (speedup 1.0000x reference)
import functools

import jax
import jax.numpy as jnp
from jax import lax
from jax.experimental import pallas as pl
from jax.experimental.pallas import tpu as pltpu
from jax.experimental.pallas import tpu_sc as plsc

_D = 768
_L = 16
_NDC = _D // _L
_C = 32
_EPS = 1e-12


def _lane_sum(x):
    lanes = lax.iota(jnp.int32, _L)
    dnums = lax.GatherDimensionNumbers(
        offset_dims=(), collapsed_slice_dims=(0,), start_index_map=(0,))
    for shift in (8, 4, 2, 1):
        perm = lanes ^ shift
        x = x + lax.gather(x, perm[:, None], dnums, (1,),
                           mode=lax.GatherScatterMode.PROMISE_IN_BOUNDS)
    return x


def _make_sc_kernel(N, S, V, T):
    info = plsc.get_sparse_core_info()
    nc, ns = info.num_cores, info.num_subcores
    nw = nc * ns
    tpw = N // nw
    nch = tpw // _C
    mesh = plsc.VectorSubcoreMesh(core_axis_name="c", subcore_axis_name="s")

    @functools.partial(
        pl.kernel,
        out_type=jax.ShapeDtypeStruct((N, _D), jnp.float32),
        mesh=mesh,
        compiler_params=pltpu.CompilerParams(needs_layout_passes=False),
        scratch_types=[
            pltpu.VMEM((_C,), jnp.int32),
            pltpu.VMEM((_C,), jnp.int32),
            pltpu.VMEM((_C, _D), jnp.float32),
            pltpu.VMEM((_C, _D), jnp.float32),
            pltpu.VMEM((_C, _D), jnp.float32),
            pltpu.VMEM((_D,), jnp.float32),
            pltpu.VMEM((_D,), jnp.float32),
            pltpu.SemaphoreType.DMA,
            pltpu.SemaphoreType.DMA,
        ],
    )
    def k(ids_hbm, seg_hbm, tok_hbm, pos_hbm, type_hbm, g_hbm, b_hbm, out_hbm,
          idx_v, seg_v, x_v, p_v, t_v, g_v, b_v, sem1, sem2):
        wid = lax.axis_index("s") * nc + lax.axis_index("c")
        base0 = wid * tpw
        pltpu.sync_copy(g_hbm, g_v)
        pltpu.sync_copy(b_hbm, b_v)

        @pl.loop(0, nch)
        def _chunk(c):
            base = base0 + c * _C
            pos_base = lax.rem(base, S)
            pltpu.sync_copy(ids_hbm.at[pl.ds(base, _C)], idx_v)
            pltpu.sync_copy(seg_hbm.at[pl.ds(base, _C)], seg_v)
            cp1 = pltpu.async_copy(tok_hbm.at[idx_v], x_v, sem1)
            cp2 = pltpu.async_copy(type_hbm.at[seg_v], t_v, sem2)
            pltpu.sync_copy(pos_hbm.at[pl.ds(pos_base, _C)], p_v)
            cp1.wait()
            cp2.wait()

            @pl.loop(0, _C)
            def _tok(t):
                acc = jnp.zeros((_L,), jnp.float32)
                acc2 = jnp.zeros((_L,), jnp.float32)
                for j in range(_NDC):
                    sl = pl.ds(j * _L, _L)
                    x = x_v[t, sl] + p_v[t, sl] + t_v[t, sl]
                    x_v[t, sl] = x
                    acc = acc + x
                    acc2 = acc2 + x * x
                mu = _lane_sum(acc) * (1.0 / _D)
                v = _lane_sum(acc2) * (1.0 / _D) - mu * mu + _EPS
                i = plsc.bitcast(v, jnp.int32)
                i = jnp.int32(0x5F3759DF) - (i >> 1)
                y = plsc.bitcast(i, jnp.float32)
                for _ in range(3):
                    y = y * (1.5 - 0.5 * v * y * y)
                for j in range(_NDC):
                    sl = pl.ds(j * _L, _L)
                    x = x_v[t, sl]
                    x_v[t, sl] = (x - mu) * y * g_v[sl] + b_v[sl]

            pltpu.sync_copy(x_v, out_hbm.at[pl.ds(base, _C)])

    return k


@jax.jit
def kernel(input_ids, segment_ids, token_table, pos_table, type_table,
           ln_gamma, ln_beta):
    B, S = input_ids.shape
    V, D = token_table.shape
    T = type_table.shape[0]
    N = B * S
    ids = input_ids.reshape(N).astype(jnp.int32)
    segs = segment_ids.reshape(N).astype(jnp.int32)
    k = _make_sc_kernel(N, S, V, T)
    out = k(ids, segs, token_table, pos_table, type_table, ln_gamma, ln_beta)
    return out.reshape(B, S, D)
